# Initial kernel scaffold; baseline (speedup 1.0000x reference)
import jax
import jax.numpy as jnp
from jax.experimental import pallas as pl


def kernel(hidden_states, ln1_w, ln1_b, ln2_w, ln2_b, Wq, bq, Wk, bk, Wv, bv, Wo, bo, Wg, w1, w3, w2, position_ids):
    raise NotImplementedError("write your pallas kernel here")


# speedup vs baseline: 1.2559x; 1.2559x over previous
import functools

import jax
import jax.numpy as jnp
from jax import lax
from jax.experimental import pallas as pl
from jax.experimental.pallas import tpu as pltpu
from jax.experimental.pallas import tpu_sc as plsc

B = 1; S = 2048; D = 2048; F = 4096; E = 8; K = 2
H = 16; HKV = 8; DH = 128; EPS = 1e-5; THETA = 10000.0

BT = 256
NQKV = H * DH + 2 * HKV * DH
NB = 1024
NT = (S * K) // BT + E
NSLOT = NT * BT
FC = 512
SCALE = 1.0 / (DH ** 0.5)


def _ln(x, w, b):
    m = jnp.mean(x, 1, keepdims=True)
    v = jnp.mean((x - m) * (x - m), 1, keepdims=True)
    return (x - m) * lax.rsqrt(v + EPS) * w + b


def _qkv_kernel(x_ref, w_ref, b_ref, lnw_ref, lnb_ref, ang_ref, o_ref):
    j = pl.program_id(1)
    h = _ln(x_ref[...], lnw_ref[...], lnb_ref[...])
    y = jnp.dot(h, w_ref[...], preferred_element_type=jnp.float32) + b_ref[...]
    c = jnp.cos(ang_ref[...])
    s = jnp.sin(ang_ref[...])
    cos = jnp.concatenate([c, c], 1)
    sin = jnp.concatenate([s, s], 1)
    ncol = NB // DH
    cos_t = jnp.concatenate([cos] * ncol, 1)
    sin_t = jnp.concatenate([sin] * ncol, 1)
    rot = jnp.concatenate(
        [jnp.concatenate([-y[:, hh * DH + DH // 2:(hh + 1) * DH],
                          y[:, hh * DH:hh * DH + DH // 2]], 1)
         for hh in range(ncol)], 1)
    yr = y * cos_t + rot * sin_t
    o_ref[...] = jnp.where(j < 3, yr, y)


def _attn_kernel(q_ref, k_ref, v_ref, o_ref):
    iq = pl.program_id(1)
    q = q_ref[...]
    s = lax.dot_general(q, k_ref[...], (((1,), (1,)), ((), ())),
                        preferred_element_type=jnp.float32) * SCALE
    rows = iq * BT + lax.broadcasted_iota(jnp.int32, (BT, S), 0)
    cols = lax.broadcasted_iota(jnp.int32, (BT, S), 1)
    s = jnp.where(cols <= rows, s, -1e9)
    m = jnp.max(s, 1, keepdims=True)
    p = jnp.exp(s - m)
    p = p / jnp.sum(p, 1, keepdims=True)
    o_ref[...] = jnp.dot(p, v_ref[...], preferred_element_type=jnp.float32)


def _oproj_kernel(a_ref, w_ref, b_ref, r_ref, o_ref):
    o_ref[...] = (r_ref[...] + b_ref[...]
                  + jnp.dot(a_ref[...], w_ref[...],
                            preferred_element_type=jnp.float32))


def _router_kernel(x_ref, lnw_ref, lnb_ref, wg_ref, x2_ref, wi_ref):
    x2 = _ln(x_ref[...], lnw_ref[...], lnb_ref[...])
    x2_ref[...] = x2
    lg = jnp.dot(x2, wg_ref[...], preferred_element_type=jnp.float32)
    lane = lax.broadcasted_iota(jnp.int32, (BT, 128), 1)
    neg = jnp.float32(-1e30)
    lg = jnp.where(lane < E, lg, neg)
    m1 = jnp.max(lg, 1, keepdims=True)
    i1 = jnp.min(jnp.where(lg == m1, lane, 1024), 1, keepdims=True)
    lg2 = jnp.where(lane == i1, neg, lg)
    m2 = jnp.max(lg2, 1, keepdims=True)
    i2 = jnp.min(jnp.where(lg2 == m2, lane, 1024), 1, keepdims=True)
    e2 = jnp.exp(m2 - m1)
    w1 = 1.0 / (1.0 + e2)
    w2 = e2 / (1.0 + e2)
    cols = [w1, w2, i1.astype(jnp.float32), i2.astype(jnp.float32)]
    out = jnp.zeros((BT, 128), jnp.float32)
    for ci, cv in enumerate(cols):
        out = jnp.where(lane == ci, cv, out)
    wi_ref[...] = out


def _rt_kernel(wi_ref, rt_ref):
    blk = wi_ref[...]
    r = lax.broadcasted_iota(jnp.int32, (BT, BT), 0)
    c = lax.broadcasted_iota(jnp.int32, (BT, BT), 1)
    eye = (r == c).astype(jnp.float32)
    t = lax.dot_general(blk, eye, (((0,), (0,)), ((), ())),
                        preferred_element_type=jnp.float32)
    rt_ref[...] = t[:8, :]


@functools.cache
def _make_sc_dispatch():
    mesh = plsc.VectorSubcoreMesh(core_axis_name="c", subcore_axis_name="s")
    return pl.kernel(
        _sc_dispatch_body,
        out_type=[
            jax.ShapeDtypeStruct((NSLOT,), jnp.int32),
            jax.ShapeDtypeStruct((K * S,), jnp.int32),
            jax.ShapeDtypeStruct((8,), jnp.int32),
        ],
        mesh=mesh,
        scratch_types=[
            pltpu.VMEM((K, S), jnp.float32),
            pltpu.VMEM((NSLOT,), jnp.int32),
            pltpu.VMEM((K * S,), jnp.int32),
            pltpu.VMEM((16,), jnp.int32),
        ],
        compiler_params=pltpu.CompilerParams(needs_layout_passes=False),
    )


def _sc_dispatch(rt):
    return _make_sc_dispatch()(rt)


def _sc_dispatch_body(rt_hbm, gtok_hbm, pos_hbm, cnt_hbm, idsf_v, gtok_v,
                      pos_v, cnt_v):
    wid = lax.axis_index("s") * 2 + lax.axis_index("c")

    @pl.when(wid == 0)
    def _():
        pltpu.sync_copy(rt_hbm.at[pl.ds(2, K)], idsf_v)
        lanes = lax.iota(jnp.int32, 16)

        def zero_body(i, _):
            gtok_v[pl.ds(i * 16, 16)] = jnp.zeros((16,), jnp.int32)
            return 0
        lax.fori_loop(0, NSLOT // 16, zero_body, 0)

        nchunk = (K * S) // 16

        def cnt_body(i, cvec):
            av = idsf_v[i // (S // 16), pl.ds((i % (S // 16)) * 16, 16)]
            a = av.astype(jnp.int32)
            for e in range(E):
                ce = jnp.sum((a == e).astype(jnp.int32))
                cvec = cvec + jnp.where(lanes == e, ce, 0)
            return cvec
        cvec = lax.fori_loop(0, nchunk, cnt_body, jnp.zeros((16,), jnp.int32))
        cnt_v[...] = cvec

        cs = [jnp.sum(jnp.where(lanes == e, cvec, 0)) for e in range(E)]
        offs = []
        o = jnp.int32(0)
        for e in range(E):
            offs.append(o)
            o = o + ((cs[e] + (BT - 1)) // BT) * BT

        def pos_body(i, carry):
            av = idsf_v[i // (S // 16), pl.ds((i % (S // 16)) * 16, 16)]
            a = av.astype(jnp.int32)
            posv = jnp.zeros((16,), jnp.int32)
            new = []
            for e in range(E):
                m = a == e
                cum = jnp.cumsum(m.astype(jnp.int32))
                rank = carry[e] + cum - 1
                posv = jnp.where(m, offs[e] + rank, posv)
                new.append(carry[e] + jnp.max(cum))
            base = i * 16
            tok = (base + lanes) % S
            plsc.store_scatter(gtok_v, [posv], tok)
            pos_v[pl.ds(base, 16)] = posv
            return tuple(new)
        lax.fori_loop(0, nchunk, pos_body,
                      tuple(jnp.int32(0) for _ in range(E)))

        pltpu.sync_copy(gtok_v, gtok_hbm)
        pltpu.sync_copy(pos_v, pos_hbm)
        pltpu.sync_copy(cnt_v.at[pl.ds(0, 8)], cnt_hbm)


_GNW = 32
_GBW = NSLOT // _GNW
_GCH = 24


@functools.cache
def _make_sc_gather():
    mesh = plsc.VectorSubcoreMesh(core_axis_name="c", subcore_axis_name="s")
    return pl.kernel(
        _sc_gather_body,
        out_type=jax.ShapeDtypeStruct((NSLOT, D), jnp.float32),
        mesh=mesh,
        scratch_types=[
            pltpu.VMEM((_GCH,), jnp.int32),
            pltpu.VMEM((_GCH, D), jnp.float32),
            pltpu.SemaphoreType.DMA,
        ],
    )


def _sc_gather(x2, gtok):
    return _make_sc_gather()(x2, gtok)


def _sc_gather_body(x2_hbm, g_hbm, xg_hbm, idx_v, rows_v, sem):
    wid = lax.axis_index("s") * 2 + lax.axis_index("c")
    base = wid * _GBW
    for ch in range(_GBW // _GCH):
        b0 = base + ch * _GCH
        pltpu.sync_copy(g_hbm.at[pl.ds(b0, _GCH)], idx_v)
        pltpu.async_copy(x2_hbm.at[idx_v], rows_v, sem).wait()
        pltpu.sync_copy(rows_v, xg_hbm.at[pl.ds(b0, _GCH)])


def _ffn_kernel(meta_ref, xg_ref, w1_ref, w3_ref, w2_ref, ys_ref, acc_ref):
    j = pl.program_id(0)
    c = pl.program_id(1)
    nlive = meta_ref[NT]

    @pl.when(j < nlive)
    def _():
        xg = xg_ref[...]
        h1 = jnp.dot(xg, w1_ref[0], preferred_element_type=jnp.float32)
        h3 = jnp.dot(xg, w3_ref[0], preferred_element_type=jnp.float32)
        hc = h1 * jax.nn.sigmoid(h1) * h3
        part = jnp.dot(hc, w2_ref[0], preferred_element_type=jnp.float32)

        @pl.when(c == 0)
        def _():
            acc_ref[...] = part

        @pl.when(c > 0)
        def _():
            acc_ref[...] = acc_ref[...] + part

        @pl.when(c == F // FC - 1)
        def _():
            ys_ref[...] = acc_ref[...]


_CTW = S // 32
_CCH = 8


@functools.cache
def _make_sc_combine():
    mesh = plsc.VectorSubcoreMesh(core_axis_name="c", subcore_axis_name="s")
    return pl.kernel(
        _sc_combine_body,
        out_type=jax.ShapeDtypeStruct((S, D), jnp.float32),
        mesh=mesh,
        scratch_types=[
            pltpu.VMEM((_CCH,), jnp.int32),
            pltpu.VMEM((_CCH,), jnp.int32),
            pltpu.VMEM((_CTW,), jnp.float32),
            pltpu.VMEM((_CTW,), jnp.float32),
            pltpu.VMEM((_CCH, D), jnp.float32),
            pltpu.VMEM((_CCH, D), jnp.float32),
            pltpu.VMEM((_CCH, D), jnp.float32),
            pltpu.SemaphoreType.DMA,
        ],
    )


def _sc_combine(h2, ys, rt, pos):
    return _make_sc_combine()(h2, ys, rt, pos)


def _sc_combine_body(h2_hbm, ys_hbm, rt_hbm, pos_hbm, out_hbm, idx0_v, idx1_v,
                     w0_v, w1_v, y0_v, y1_v, acc_v, sem):
    wid = lax.axis_index("s") * 2 + lax.axis_index("c")
    tbase = wid * _CTW
    pltpu.sync_copy(rt_hbm.at[0, pl.ds(tbase, _CTW)], w0_v)
    pltpu.sync_copy(rt_hbm.at[1, pl.ds(tbase, _CTW)], w1_v)
    for ch in range(_CTW // _CCH):
        t0 = tbase + ch * _CCH
        pltpu.sync_copy(pos_hbm.at[pl.ds(t0, _CCH)], idx0_v)
        pltpu.sync_copy(pos_hbm.at[pl.ds(S + t0, _CCH)], idx1_v)
        pltpu.async_copy(ys_hbm.at[idx0_v], y0_v, sem).wait()
        pltpu.async_copy(ys_hbm.at[idx1_v], y1_v, sem).wait()
        pltpu.sync_copy(h2_hbm.at[pl.ds(t0, _CCH)], acc_v)
        wbase = (ch * _CCH // 16) * 16
        wv0 = w0_v[pl.ds(wbase, 16)]
        wv1 = w1_v[pl.ds(wbase, 16)]
        for r in range(_CCH):
            a = wv0[(ch * _CCH + r) - wbase]
            b = wv1[(ch * _CCH + r) - wbase]

            def dbody(dd, _, _r=r, _a=a, _b=b):
                sl = pl.ds(dd * 16, 16)
                acc_v[_r, sl] = (acc_v[_r, sl] + _a * y0_v[_r, sl]
                                 + _b * y1_v[_r, sl])
                return 0
            lax.fori_loop(0, D // 16, dbody, 0)
        pltpu.sync_copy(acc_v, out_hbm.at[pl.ds(t0, _CCH)])


def kernel(hidden_states, ln1_w, ln1_b, ln2_w, ln2_b, Wq, bq, Wk, bk, Wv, bv,
           Wo, bo, Wg, w1, w3, w2, position_ids):
    x = hidden_states.reshape(S, D)
    lnw1 = ln1_w.reshape(1, D)
    lnb1 = ln1_b.reshape(1, D)
    lnw2 = ln2_w.reshape(1, D)
    lnb2 = ln2_b.reshape(1, D)

    inv = 1.0 / (THETA ** (jnp.arange(0, DH, 2, dtype=jnp.float32) / DH))
    ang = position_ids.reshape(S, 1).astype(jnp.float32) * inv[None, :]

    Wqkv = jnp.concatenate([Wq, Wk, Wv], 1)
    bqkv = jnp.concatenate([bq, bk, bv]).reshape(1, NQKV)

    qkv = pl.pallas_call(
        _qkv_kernel,
        grid=(S // BT, NQKV // NB),
        in_specs=[
            pl.BlockSpec((BT, D), lambda i, j: (i, 0)),
            pl.BlockSpec((D, NB), lambda i, j: (0, j)),
            pl.BlockSpec((1, NB), lambda i, j: (0, j)),
            pl.BlockSpec((1, D), lambda i, j: (0, 0)),
            pl.BlockSpec((1, D), lambda i, j: (0, 0)),
            pl.BlockSpec((BT, DH // 2), lambda i, j: (i, 0)),
        ],
        out_specs=pl.BlockSpec((BT, NB), lambda i, j: (i, j)),
        out_shape=jax.ShapeDtypeStruct((S, NQKV), jnp.float32),
    )(x, Wqkv, bqkv, lnw1, lnb1, ang)

    q2d = qkv[:, :H * DH]
    k2d = qkv[:, H * DH:H * DH + HKV * DH]
    v2d = qkv[:, H * DH + HKV * DH:]

    rep = H // HKV
    ao = pl.pallas_call(
        _attn_kernel,
        grid=(H, S // BT),
        in_specs=[
            pl.BlockSpec((BT, DH), lambda h, i: (i, h)),
            pl.BlockSpec((S, DH), lambda h, i: (0, h // rep)),
            pl.BlockSpec((S, DH), lambda h, i: (0, h // rep)),
        ],
        out_specs=pl.BlockSpec((BT, DH), lambda h, i: (i, h)),
        out_shape=jax.ShapeDtypeStruct((S, H * DH), jnp.float32),
    )(q2d, k2d, v2d)

    h2 = pl.pallas_call(
        _oproj_kernel,
        grid=(S // BT, D // NB),
        in_specs=[
            pl.BlockSpec((BT, H * DH), lambda i, j: (i, 0)),
            pl.BlockSpec((H * DH, NB), lambda i, j: (0, j)),
            pl.BlockSpec((1, NB), lambda i, j: (0, j)),
            pl.BlockSpec((BT, NB), lambda i, j: (i, j)),
        ],
        out_specs=pl.BlockSpec((BT, NB), lambda i, j: (i, j)),
        out_shape=jax.ShapeDtypeStruct((S, D), jnp.float32),
    )(ao, Wo, bo.reshape(1, D), x)

    Wgp = jnp.pad(Wg, ((0, 0), (0, 128 - E)))
    x2, wi = pl.pallas_call(
        _router_kernel,
        grid=(S // BT,),
        in_specs=[
            pl.BlockSpec((BT, D), lambda i: (i, 0)),
            pl.BlockSpec((1, D), lambda i: (0, 0)),
            pl.BlockSpec((1, D), lambda i: (0, 0)),
            pl.BlockSpec((D, 128), lambda i: (0, 0)),
        ],
        out_specs=[
            pl.BlockSpec((BT, D), lambda i: (i, 0)),
            pl.BlockSpec((BT, 128), lambda i: (i, 0)),
        ],
        out_shape=[
            jax.ShapeDtypeStruct((S, D), jnp.float32),
            jax.ShapeDtypeStruct((S, 128), jnp.float32),
        ],
    )(h2, lnw2, lnb2, Wgp)

    rt = pl.pallas_call(
        _rt_kernel,
        grid=(S // BT,),
        in_specs=[pl.BlockSpec((BT, 128), lambda i: (i, 0))],
        out_specs=pl.BlockSpec((8, BT), lambda i: (0, i)),
        out_shape=jax.ShapeDtypeStruct((8, S), jnp.float32),
    )(wi)

    gtok, pos, counts = _sc_dispatch(rt)

    ntiles = (counts + (BT - 1)) // BT
    cum = jnp.cumsum(ntiles)
    nlive = cum[E - 1]
    jj = jnp.arange(NT, dtype=jnp.int32)
    te = jnp.minimum(jnp.sum((jj[:, None] >= cum[None, :]).astype(jnp.int32),
                             1), E - 1)
    meta = jnp.concatenate([te, nlive[None].astype(jnp.int32),
                            jnp.zeros((1,), jnp.int32)])

    xg = _sc_gather(x2, gtok)

    ys = pl.pallas_call(
        _ffn_kernel,
        grid_spec=pltpu.PrefetchScalarGridSpec(
            num_scalar_prefetch=1,
            grid=(NT, F // FC),
            in_specs=[
                pl.BlockSpec((BT, D), lambda j, c, m: (j, 0)),
                pl.BlockSpec((1, D, FC), lambda j, c, m: (m[j], 0, c)),
                pl.BlockSpec((1, D, FC), lambda j, c, m: (m[j], 0, c)),
                pl.BlockSpec((1, FC, D), lambda j, c, m: (m[j], c, 0)),
            ],
            out_specs=pl.BlockSpec((BT, D), lambda j, c, m: (j, 0)),
            scratch_shapes=[pltpu.VMEM((BT, D), jnp.float32)],
        ),
        out_shape=jax.ShapeDtypeStruct((NSLOT, D), jnp.float32),
    )(meta, xg, w1, w3, w2)

    out = _sc_combine(h2, ys, rt, pos)
    return out.reshape(B, S, D)


# speedup vs baseline: 1.3761x; 1.0957x over previous
import functools

import jax
import jax.numpy as jnp
from jax import lax
from jax.experimental import pallas as pl
from jax.experimental.pallas import tpu as pltpu
from jax.experimental.pallas import tpu_sc as plsc

B = 1; S = 2048; D = 2048; F = 4096; E = 8; K = 2
H = 16; HKV = 8; DH = 128; EPS = 1e-5; THETA = 10000.0

BT = 256
NQKV = H * DH + 2 * HKV * DH
NB = 1024
FBT = 512
NT = (S * K) // FBT + E
NSLOT = NT * FBT
FC = 512
SCALE = 1.0 / (DH ** 0.5)


def _ln(x, w, b):
    m = jnp.mean(x, 1, keepdims=True)
    v = jnp.mean((x - m) * (x - m), 1, keepdims=True)
    return (x - m) * lax.rsqrt(v + EPS) * w + b


def _qkv_kernel(x_ref, w_ref, b_ref, lnw_ref, lnb_ref, ang_ref, o_ref):
    j = pl.program_id(0)
    h = _ln(x_ref[...], lnw_ref[...], lnb_ref[...])
    y = jnp.dot(h, w_ref[...], preferred_element_type=jnp.float32) + b_ref[...]
    c = jnp.cos(ang_ref[...])
    s = jnp.sin(ang_ref[...])
    cos = jnp.concatenate([c, c], 1)
    sin = jnp.concatenate([s, s], 1)
    ncol = NB // DH
    cos_t = jnp.concatenate([cos] * ncol, 1)
    sin_t = jnp.concatenate([sin] * ncol, 1)
    rot = jnp.concatenate(
        [jnp.concatenate([-y[:, hh * DH + DH // 2:(hh + 1) * DH],
                          y[:, hh * DH:hh * DH + DH // 2]], 1)
         for hh in range(ncol)], 1)
    yr = y * cos_t + rot * sin_t
    o_ref[...] = jnp.where(j < 3, yr, y)


def _attn_kernel(q_ref, k_ref, v_ref, o_ref):
    iq = pl.program_id(1)
    q = q_ref[...]
    s = lax.dot_general(q, k_ref[...], (((1,), (1,)), ((), ())),
                        preferred_element_type=jnp.float32) * SCALE
    rows = iq * BT + lax.broadcasted_iota(jnp.int32, (BT, S), 0)
    cols = lax.broadcasted_iota(jnp.int32, (BT, S), 1)
    s = jnp.where(cols <= rows, s, -1e9)
    m = jnp.max(s, 1, keepdims=True)
    p = jnp.exp(s - m)
    p = p / jnp.sum(p, 1, keepdims=True)
    o_ref[...] = jnp.dot(p, v_ref[...], preferred_element_type=jnp.float32)


def _oproj_kernel(a_ref, w_ref, b_ref, r_ref, o_ref):
    o_ref[...] = (r_ref[...] + b_ref[...]
                  + jnp.dot(a_ref[...], w_ref[...],
                            preferred_element_type=jnp.float32))


def _router_kernel(x_ref, lnw_ref, lnb_ref, wg_ref, x2_ref, wi_ref):
    x2 = _ln(x_ref[...], lnw_ref[...], lnb_ref[...])
    x2_ref[...] = x2
    lg = jnp.dot(x2, wg_ref[...], preferred_element_type=jnp.float32)
    lane = lax.broadcasted_iota(jnp.int32, (BT, 128), 1)
    neg = jnp.float32(-1e30)
    lg = jnp.where(lane < E, lg, neg)
    m1 = jnp.max(lg, 1, keepdims=True)
    i1 = jnp.min(jnp.where(lg == m1, lane, 1024), 1, keepdims=True)
    lg2 = jnp.where(lane == i1, neg, lg)
    m2 = jnp.max(lg2, 1, keepdims=True)
    i2 = jnp.min(jnp.where(lg2 == m2, lane, 1024), 1, keepdims=True)
    e2 = jnp.exp(m2 - m1)
    w1 = 1.0 / (1.0 + e2)
    w2 = e2 / (1.0 + e2)
    cols = [w1, w2, i1.astype(jnp.float32), i2.astype(jnp.float32)]
    out = jnp.zeros((BT, 128), jnp.float32)
    for ci, cv in enumerate(cols):
        out = jnp.where(lane == ci, cv, out)
    wi_ref[...] = out


def _rt_kernel(wi_ref, rt_ref):
    blk = wi_ref[...]
    r = lax.broadcasted_iota(jnp.int32, (BT, BT), 0)
    c = lax.broadcasted_iota(jnp.int32, (BT, BT), 1)
    eye = (r == c).astype(jnp.float32)
    t = lax.dot_general(blk, eye, (((0,), (0,)), ((), ())),
                        preferred_element_type=jnp.float32)
    rt_ref[...] = t[:8, :]


@functools.cache
def _make_sc_dispatch():
    mesh = plsc.VectorSubcoreMesh(core_axis_name="c", subcore_axis_name="s")
    return pl.kernel(
        _sc_dispatch_body,
        out_type=[
            jax.ShapeDtypeStruct((NSLOT,), jnp.int32),
            jax.ShapeDtypeStruct((K * S,), jnp.int32),
            jax.ShapeDtypeStruct((8,), jnp.int32),
        ],
        mesh=mesh,
        scratch_types=[
            pltpu.VMEM((K, S), jnp.float32),
            pltpu.VMEM((NSLOT,), jnp.int32),
            pltpu.VMEM((K * S,), jnp.int32),
            pltpu.VMEM((16,), jnp.int32),
        ],
        compiler_params=pltpu.CompilerParams(needs_layout_passes=False),
    )


def _sc_dispatch(rt):
    return _make_sc_dispatch()(rt)


def _sc_dispatch_body(rt_hbm, gtok_hbm, pos_hbm, cnt_hbm, idsf_v, gtok_v,
                      pos_v, cnt_v):
    wid = lax.axis_index("s") * 2 + lax.axis_index("c")

    @pl.when(wid == 0)
    def _():
        pltpu.sync_copy(rt_hbm.at[pl.ds(2, K)], idsf_v)
        lanes = lax.iota(jnp.int32, 16)

        def zero_body(i, _):
            gtok_v[pl.ds(i * 16, 16)] = jnp.zeros((16,), jnp.int32)
            return 0
        lax.fori_loop(0, NSLOT // 16, zero_body, 0)

        nchunk = (K * S) // 16

        def cnt_body(i, cvec):
            av = idsf_v[i // (S // 16), pl.ds((i % (S // 16)) * 16, 16)]
            a = av.astype(jnp.int32)
            for e in range(E):
                ce = jnp.sum((a == e).astype(jnp.int32))
                cvec = cvec + jnp.where(lanes == e, ce, 0)
            return cvec
        cvec = lax.fori_loop(0, nchunk, cnt_body, jnp.zeros((16,), jnp.int32))
        cnt_v[...] = cvec

        cs = [jnp.sum(jnp.where(lanes == e, cvec, 0)) for e in range(E)]
        offs = []
        o = jnp.int32(0)
        for e in range(E):
            offs.append(o)
            o = o + ((cs[e] + (FBT - 1)) // FBT) * FBT

        def pos_body(i, carry):
            av = idsf_v[i // (S // 16), pl.ds((i % (S // 16)) * 16, 16)]
            a = av.astype(jnp.int32)
            posv = jnp.zeros((16,), jnp.int32)
            new = []
            for e in range(E):
                m = a == e
                cum = jnp.cumsum(m.astype(jnp.int32))
                rank = carry[e] + cum - 1
                posv = jnp.where(m, offs[e] + rank, posv)
                new.append(carry[e] + jnp.max(cum))
            base = i * 16
            tok = (base + lanes) % S
            plsc.store_scatter(gtok_v, [posv], tok)
            pos_v[pl.ds(base, 16)] = posv
            return tuple(new)
        lax.fori_loop(0, nchunk, pos_body,
                      tuple(jnp.int32(0) for _ in range(E)))

        pltpu.sync_copy(gtok_v, gtok_hbm)
        pltpu.sync_copy(pos_v, pos_hbm)
        pltpu.sync_copy(cnt_v.at[pl.ds(0, 8)], cnt_hbm)


_GNW = 32
_GBW = NSLOT // _GNW
_GCH = 16
_GNC = _GBW // _GCH


@functools.cache
def _make_sc_gather():
    mesh = plsc.VectorSubcoreMesh(core_axis_name="c", subcore_axis_name="s")
    return pl.kernel(
        _sc_gather_body,
        out_type=jax.ShapeDtypeStruct((NSLOT, D), jnp.float32),
        mesh=mesh,
        scratch_types=[
            pltpu.VMEM((_GCH,), jnp.int32),
            pltpu.VMEM((_GCH,), jnp.int32),
            pltpu.VMEM((_GCH, D), jnp.float32),
            pltpu.VMEM((_GCH, D), jnp.float32),
            pltpu.SemaphoreType.DMA,
            pltpu.SemaphoreType.DMA,
            pltpu.SemaphoreType.DMA,
            pltpu.SemaphoreType.DMA,
        ],
    )


def _sc_gather(x2, gtok):
    return _make_sc_gather()(x2, gtok)


def _sc_gather_body(x2_hbm, g_hbm, xg_hbm, i0, i1, r0, r1, gs0, gs1, ss0,
                    ss1):
    wid = lax.axis_index("s") * 2 + lax.axis_index("c")
    base = wid * _GBW
    idx = (i0, i1)
    rows = (r0, r1)
    gsem = (gs0, gs1)
    ssem = (ss0, ss1)
    gcp = [None, None]
    scp = [None, None]
    pltpu.sync_copy(g_hbm.at[pl.ds(base, _GCH)], i0)
    gcp[0] = pltpu.async_copy(x2_hbm.at[i0], r0, gs0)
    for ch in range(_GNC):
        b = ch & 1
        nb = 1 - b
        if ch + 1 < _GNC:
            if ch >= 1:
                scp[nb].wait()
            pltpu.sync_copy(g_hbm.at[pl.ds(base + (ch + 1) * _GCH, _GCH)],
                            idx[nb])
            gcp[nb] = pltpu.async_copy(x2_hbm.at[idx[nb]], rows[nb], gsem[nb])
        gcp[b].wait()
        scp[b] = pltpu.async_copy(rows[b], xg_hbm.at[pl.ds(base + ch * _GCH,
                                                           _GCH)], ssem[b])
    scp[0].wait()
    scp[1].wait()


def _ffn_kernel(meta_ref, xg_ref, w1_ref, w3_ref, w2_ref, ys_ref, acc_ref):
    j = pl.program_id(0)
    c = pl.program_id(1)
    nlive = meta_ref[NT]

    @pl.when(j < nlive)
    def _():
        xg = xg_ref[...]
        h1 = jnp.dot(xg, w1_ref[0], preferred_element_type=jnp.float32)
        h3 = jnp.dot(xg, w3_ref[0], preferred_element_type=jnp.float32)
        hc = h1 * jax.nn.sigmoid(h1) * h3
        part = jnp.dot(hc, w2_ref[0], preferred_element_type=jnp.float32)

        @pl.when(c == 0)
        def _():
            acc_ref[...] = part

        @pl.when(c > 0)
        def _():
            acc_ref[...] = acc_ref[...] + part

        @pl.when(c == F // FC - 1)
        def _():
            ys_ref[...] = acc_ref[...]


_CTW = S // 32
_CCH = 8
_CNC = _CTW // _CCH


@functools.cache
def _make_sc_combine():
    mesh = plsc.VectorSubcoreMesh(core_axis_name="c", subcore_axis_name="s")
    return pl.kernel(
        _sc_combine_body,
        out_type=jax.ShapeDtypeStruct((S, D), jnp.float32),
        mesh=mesh,
        scratch_types=[
            pltpu.VMEM((_CCH,), jnp.int32),
            pltpu.VMEM((_CCH,), jnp.int32),
            pltpu.VMEM((_CCH,), jnp.int32),
            pltpu.VMEM((_CCH,), jnp.int32),
            pltpu.VMEM((_CTW,), jnp.float32),
            pltpu.VMEM((_CTW,), jnp.float32),
            pltpu.VMEM((_CCH, D), jnp.float32),
            pltpu.VMEM((_CCH, D), jnp.float32),
            pltpu.VMEM((_CCH, D), jnp.float32),
            pltpu.VMEM((_CCH, D), jnp.float32),
            pltpu.VMEM((_CCH, D), jnp.float32),
            pltpu.VMEM((_CCH, D), jnp.float32),
            pltpu.SemaphoreType.DMA,
            pltpu.SemaphoreType.DMA,
            pltpu.SemaphoreType.DMA,
            pltpu.SemaphoreType.DMA,
            pltpu.SemaphoreType.DMA,
            pltpu.SemaphoreType.DMA,
            pltpu.SemaphoreType.DMA,
            pltpu.SemaphoreType.DMA,
        ],
    )


def _sc_combine(h2, ys, rt, pos):
    return _make_sc_combine()(h2, ys, rt, pos)


def _sc_combine_body(h2_hbm, ys_hbm, rt_hbm, pos_hbm, out_hbm,
                     i0a, i0b, i1a, i1b, w0_v, w1_v,
                     y0a, y0b, y1a, y1b, acca, accb,
                     g0a, g0b, g1a, g1b, ha, hb, sa, sb):
    wid = lax.axis_index("s") * 2 + lax.axis_index("c")
    tbase = wid * _CTW
    idx0 = (i0a, i0b)
    idx1 = (i1a, i1b)
    y0 = (y0a, y0b)
    y1 = (y1a, y1b)
    acc = (acca, accb)
    gs0 = (g0a, g0b)
    gs1 = (g1a, g1b)
    hs = (ha, hb)
    ss = (sa, sb)
    cp0 = [None, None]
    cp1 = [None, None]
    cph = [None, None]
    scp = [None, None]
    pltpu.sync_copy(rt_hbm.at[0, pl.ds(tbase, _CTW)], w0_v)
    pltpu.sync_copy(rt_hbm.at[1, pl.ds(tbase, _CTW)], w1_v)

    def issue(ch, b):
        t0 = tbase + ch * _CCH
        pltpu.sync_copy(pos_hbm.at[pl.ds(t0, _CCH)], idx0[b])
        pltpu.sync_copy(pos_hbm.at[pl.ds(S + t0, _CCH)], idx1[b])
        cp0[b] = pltpu.async_copy(ys_hbm.at[idx0[b]], y0[b], gs0[b])
        cp1[b] = pltpu.async_copy(ys_hbm.at[idx1[b]], y1[b], gs1[b])
        cph[b] = pltpu.async_copy(h2_hbm.at[pl.ds(t0, _CCH)], acc[b], hs[b])

    issue(0, 0)
    for ch in range(_CNC):
        b = ch & 1
        nb = 1 - b
        if ch + 1 < _CNC:
            if ch >= 1:
                scp[nb].wait()
            issue(ch + 1, nb)
        cp0[b].wait()
        cp1[b].wait()
        cph[b].wait()
        wbase = (ch * _CCH // 16) * 16
        wv0 = w0_v[pl.ds(wbase, 16)]
        wv1 = w1_v[pl.ds(wbase, 16)]
        wa = [wv0[ch * _CCH + r - wbase] for r in range(_CCH)]
        wb = [wv1[ch * _CCH + r - wbase] for r in range(_CCH)]
        a_v = acc[b]
        y0_v = y0[b]
        y1_v = y1[b]

        def dbody(dd, _):
            sl = pl.ds(dd * 16, 16)
            for r in range(_CCH):
                a_v[r, sl] = (a_v[r, sl] + wa[r] * y0_v[r, sl]
                              + wb[r] * y1_v[r, sl])
            return 0
        lax.fori_loop(0, D // 16, dbody, 0)
        scp[b] = pltpu.async_copy(acc[b], out_hbm.at[pl.ds(tbase + ch * _CCH,
                                                           _CCH)], ss[b])
    scp[0].wait()
    scp[1].wait()


def kernel(hidden_states, ln1_w, ln1_b, ln2_w, ln2_b, Wq, bq, Wk, bk, Wv, bv,
           Wo, bo, Wg, w1, w3, w2, position_ids):
    x = hidden_states.reshape(S, D)
    lnw1 = ln1_w.reshape(1, D)
    lnb1 = ln1_b.reshape(1, D)
    lnw2 = ln2_w.reshape(1, D)
    lnb2 = ln2_b.reshape(1, D)

    inv = 1.0 / (THETA ** (jnp.arange(0, DH, 2, dtype=jnp.float32) / DH))
    ang = position_ids.reshape(S, 1).astype(jnp.float32) * inv[None, :]

    Wqkv = jnp.concatenate([Wq, Wk, Wv], 1)
    bqkv = jnp.concatenate([bq, bk, bv]).reshape(1, NQKV)

    qkv = pl.pallas_call(
        _qkv_kernel,
        grid=(NQKV // NB, S // BT),
        in_specs=[
            pl.BlockSpec((BT, D), lambda j, i: (i, 0)),
            pl.BlockSpec((D, NB), lambda j, i: (0, j)),
            pl.BlockSpec((1, NB), lambda j, i: (0, j)),
            pl.BlockSpec((1, D), lambda j, i: (0, 0)),
            pl.BlockSpec((1, D), lambda j, i: (0, 0)),
            pl.BlockSpec((BT, DH // 2), lambda j, i: (i, 0)),
        ],
        out_specs=pl.BlockSpec((BT, NB), lambda j, i: (i, j)),
        out_shape=jax.ShapeDtypeStruct((S, NQKV), jnp.float32),
    )(x, Wqkv, bqkv, lnw1, lnb1, ang)

    q2d = qkv[:, :H * DH]
    k2d = qkv[:, H * DH:H * DH + HKV * DH]
    v2d = qkv[:, H * DH + HKV * DH:]

    rep = H // HKV
    ao = pl.pallas_call(
        _attn_kernel,
        grid=(H, S // BT),
        in_specs=[
            pl.BlockSpec((BT, DH), lambda h, i: (i, h)),
            pl.BlockSpec((S, DH), lambda h, i: (0, h // rep)),
            pl.BlockSpec((S, DH), lambda h, i: (0, h // rep)),
        ],
        out_specs=pl.BlockSpec((BT, DH), lambda h, i: (i, h)),
        out_shape=jax.ShapeDtypeStruct((S, H * DH), jnp.float32),
    )(q2d, k2d, v2d)

    h2 = pl.pallas_call(
        _oproj_kernel,
        grid=(D // NB, S // BT),
        in_specs=[
            pl.BlockSpec((BT, H * DH), lambda j, i: (i, 0)),
            pl.BlockSpec((H * DH, NB), lambda j, i: (0, j)),
            pl.BlockSpec((1, NB), lambda j, i: (0, j)),
            pl.BlockSpec((BT, NB), lambda j, i: (i, j)),
        ],
        out_specs=pl.BlockSpec((BT, NB), lambda j, i: (i, j)),
        out_shape=jax.ShapeDtypeStruct((S, D), jnp.float32),
    )(ao, Wo, bo.reshape(1, D), x)

    Wgp = jnp.pad(Wg, ((0, 0), (0, 128 - E)))
    x2, wi = pl.pallas_call(
        _router_kernel,
        grid=(S // BT,),
        in_specs=[
            pl.BlockSpec((BT, D), lambda i: (i, 0)),
            pl.BlockSpec((1, D), lambda i: (0, 0)),
            pl.BlockSpec((1, D), lambda i: (0, 0)),
            pl.BlockSpec((D, 128), lambda i: (0, 0)),
        ],
        out_specs=[
            pl.BlockSpec((BT, D), lambda i: (i, 0)),
            pl.BlockSpec((BT, 128), lambda i: (i, 0)),
        ],
        out_shape=[
            jax.ShapeDtypeStruct((S, D), jnp.float32),
            jax.ShapeDtypeStruct((S, 128), jnp.float32),
        ],
    )(h2, lnw2, lnb2, Wgp)

    rt = pl.pallas_call(
        _rt_kernel,
        grid=(S // BT,),
        in_specs=[pl.BlockSpec((BT, 128), lambda i: (i, 0))],
        out_specs=pl.BlockSpec((8, BT), lambda i: (0, i)),
        out_shape=jax.ShapeDtypeStruct((8, S), jnp.float32),
    )(wi)

    gtok, pos, counts = _sc_dispatch(rt)

    ntiles = (counts + (FBT - 1)) // FBT
    cum = jnp.cumsum(ntiles)
    nlive = cum[E - 1]
    jj = jnp.arange(NT, dtype=jnp.int32)
    te = jnp.minimum(jnp.sum((jj[:, None] >= cum[None, :]).astype(jnp.int32),
                             1), E - 1)
    meta = jnp.concatenate([te, nlive[None].astype(jnp.int32),
                            jnp.zeros((1,), jnp.int32)])

    xg = _sc_gather(x2, gtok)

    ys = pl.pallas_call(
        _ffn_kernel,
        grid_spec=pltpu.PrefetchScalarGridSpec(
            num_scalar_prefetch=1,
            grid=(NT, F // FC),
            in_specs=[
                pl.BlockSpec((FBT, D), lambda j, c, m: (j, 0)),
                pl.BlockSpec((1, D, FC), lambda j, c, m: (m[j], 0, c)),
                pl.BlockSpec((1, D, FC), lambda j, c, m: (m[j], 0, c)),
                pl.BlockSpec((1, FC, D), lambda j, c, m: (m[j], c, 0)),
            ],
            out_specs=pl.BlockSpec((FBT, D), lambda j, c, m: (j, 0)),
            scratch_shapes=[pltpu.VMEM((FBT, D), jnp.float32)],
        ),
        out_shape=jax.ShapeDtypeStruct((NSLOT, D), jnp.float32),
    )(meta, xg, w1, w3, w2)

    out = _sc_combine(h2, ys, rt, pos)
    return out.reshape(B, S, D)
